# baseline (speedup 1.0000x reference)
import functools

import jax
import jax.numpy as jnp
from jax.experimental import pallas as pl
from jax.experimental.pallas import tpu as pltpu

_EPS = 1e-6


def _grn_phased_kernel(x_ref, gamma_ref, beta_ref, o_ref,
                       xs_ref, ssq_ref, scale_ref, *, inv_d, n_t, t_tile):
    t = pl.program_id(1)

    @pl.when(t < n_t)
    def _load_and_accumulate():
        xt = x_ref[...]
        xs_ref[:, pl.ds(t * t_tile, t_tile), :] = xt
        part = jnp.sum(xt * xt, axis=1, keepdims=True)

        @pl.when(t == 0)
        def _():
            ssq_ref[...] = part

        @pl.when(t > 0)
        def _():
            ssq_ref[...] += part

    @pl.when(t == n_t - 1)
    def _finalize_scale():
        gx = jnp.sqrt(ssq_ref[...])
        mean = jnp.sum(gx, axis=-1, keepdims=True) * inv_d
        scale_ref[...] = gamma_ref[...] * (gx / (mean + _EPS)) + 1.0

    @pl.when(t >= n_t)
    def _apply():
        j = t - n_t
        xt = xs_ref[:, pl.ds(j * t_tile, t_tile), :]
        o_ref[...] = xt * scale_ref[...] + beta_ref[...]


def kernel(x, gamma, beta):
    B, T, D = x.shape
    g = gamma.reshape(1, 1, D).astype(jnp.float32)
    b = beta.reshape(1, 1, D).astype(jnp.float32)

    Bb = 2
    n_t = 2
    Tt = T // n_t

    grid = (B // Bb, 2 * n_t)

    return pl.pallas_call(
        functools.partial(_grn_phased_kernel, inv_d=1.0 / D, n_t=n_t,
                          t_tile=Tt),
        out_shape=jax.ShapeDtypeStruct((B, T, D), x.dtype),
        grid=grid,
        in_specs=[
            pl.BlockSpec((Bb, Tt, D),
                         lambda i, t: (i, jnp.minimum(t, n_t - 1), 0)),
            pl.BlockSpec((1, 1, D), lambda i, t: (0, 0, 0)),
            pl.BlockSpec((1, 1, D), lambda i, t: (0, 0, 0)),
        ],
        out_specs=pl.BlockSpec((Bb, Tt, D),
                               lambda i, t: (i, jnp.maximum(t - n_t, 0), 0)),
        scratch_shapes=[
            pltpu.VMEM((Bb, T, D), jnp.float32),
            pltpu.VMEM((Bb, 1, D), jnp.float32),
            pltpu.VMEM((Bb, 1, D), jnp.float32),
        ],
        compiler_params=pltpu.CompilerParams(
            dimension_semantics=("parallel", "arbitrary"),
            vmem_limit_bytes=48 << 20,
        ),
    )(x.astype(jnp.float32), g, b)


# speedup vs baseline: 1.3676x; 1.3676x over previous
import functools

import jax
import jax.numpy as jnp
from jax.experimental import pallas as pl
from jax.experimental.pallas import tpu as pltpu

_EPS = 1e-6
_CH = 8


def _grn_kernel(x_ref, gamma_ref, beta_ref, o_ref, *, inv_d):
    bb, t, d = x_ref.shape
    n_chunks = t // _CH

    acc = jnp.zeros((bb, _CH, d), jnp.float32)
    for k in range(n_chunks):
        c = x_ref[:, k * _CH:(k + 1) * _CH, :]
        acc += c * c
    ssq = jnp.sum(acc, axis=1, keepdims=True)

    gx = jnp.sqrt(ssq)
    mean = jnp.sum(gx, axis=-1, keepdims=True) * inv_d
    scale = gamma_ref[...] * (gx / (mean + _EPS)) + 1.0
    beta = beta_ref[...]

    for k in range(n_chunks):
        sl = pl.ds(k * _CH, _CH)
        o_ref[:, sl, :] = x_ref[:, sl, :] * scale + beta


def kernel(x, gamma, beta):
    B, T, D = x.shape
    g = gamma.reshape(1, 1, D).astype(jnp.float32)
    b = beta.reshape(1, 1, D).astype(jnp.float32)

    Bb = 2
    grid = (B // Bb,)

    return pl.pallas_call(
        functools.partial(_grn_kernel, inv_d=1.0 / D),
        out_shape=jax.ShapeDtypeStruct((B, T, D), x.dtype),
        grid=grid,
        in_specs=[
            pl.BlockSpec((Bb, T, D), lambda i: (i, 0, 0)),
            pl.BlockSpec((1, 1, D), lambda i: (0, 0, 0)),
            pl.BlockSpec((1, 1, D), lambda i: (0, 0, 0)),
        ],
        out_specs=pl.BlockSpec((Bb, T, D), lambda i: (i, 0, 0)),
        compiler_params=pltpu.CompilerParams(
            dimension_semantics=("parallel",),
            vmem_limit_bytes=48 << 20,
        ),
    )(x.astype(jnp.float32), g, b)
